# baseline (speedup 1.0000x reference)
import functools

import jax
import jax.numpy as jnp
from jax import lax
from jax.experimental import pallas as pl
from jax.experimental.pallas import tpu as pltpu
from jax.experimental.pallas import tpu_sc as plsc

_H, _W, _N = 512, 512, 100
_CW = 128
_CPP = _W // _CW
_PPW = _H // 32
_NCHUNK = _PPW * _CPP
_OFFS = (0, 16, 32, 48, 64, 80, 84)

_BIG = 1 << 20


def _sc_body(data_hbm, rois_hbm, out_hbm,
             inb0, inb1, outb0, outb1, roisv,
             isem0, isem1, osem0, osem1):
    nc = 2
    wid = lax.axis_index("s") * nc + lax.axis_index("c")
    p0 = wid * _PPW

    inb = (inb0, inb1)
    outb = (outb0, outb1)
    isem = (isem0, isem1)
    osem = (osem0, osem1)

    pltpu.sync_copy(rois_hbm, roisv)

    def chunk_plane(k):
        return p0 + k // _CPP

    def chunk_w0(k):
        return (k % _CPP) * _CW

    def in_copy(k, b):
        return pltpu.make_async_copy(
            data_hbm.at[chunk_plane(k), pl.ds(chunk_w0(k), _CW)],
            inb[b], isem[b])

    def out_copy(k, b):
        return pltpu.make_async_copy(
            outb[b], out_hbm.at[chunk_plane(k), pl.ds(chunk_w0(k), _CW)],
            osem[b])

    los = []
    rngs = []
    y1s = []
    y2s = []
    for g, off in enumerate(_OFFS):
        x1 = roisv[0, pl.ds(off, 16)]
        y1 = roisv[1, pl.ds(off, 16)]
        x2 = roisv[2, pl.ds(off, 16)]
        y2 = roisv[3, pl.ds(off, 16)]
        x1t = x1.astype(jnp.int32)
        lo = x1t + jnp.where(x1 > x1t.astype(jnp.float32), 1, 0)
        hi = x2.astype(jnp.int32)
        los.append(lo)
        rngs.append(hi - lo)
        y1s.append(y1)
        y2s.append(y2)

    in_copy(0, 0).start()
    in_copy(1, 1).start()

    def superstep(s, carry):
        for b in range(2):
            k = s * 2 + b
            hf = chunk_plane(k).astype(jnp.float32)
            w0 = chunk_w0(k)

            lo_eff = []
            rng_u = []
            for g in range(len(_OFFS)):
                act = ((hf >= y1s[g]) & (hf <= y2s[g])
                       & (rngs[g] >= 0))
                lo_eff.append(jnp.where(act, los[g], _BIG))
                rng_u.append(rngs[g].astype(jnp.uint32))

            in_copy(k, b).wait()

            @pl.when(s >= 1)
            def _():
                out_copy(k - 2, b).wait()

            def row_body(t, carry2):
                for sub in range(8):
                    w = t * 8 + sub
                    wg = w0 + w
                    for g, off in enumerate(_OFFS):
                        d = inb[b][w, pl.ds(off, 16)]
                        dist = (wg - lo_eff[g]).astype(jnp.uint32)
                        m = dist <= rng_u[g]
                        outb[b][w, pl.ds(off, 16)] = jnp.where(m, d, 0.0)
                return carry2

            lax.fori_loop(0, _CW // 8, row_body, 0)

            out_copy(k, b).start()

            @pl.when(k + 2 < _NCHUNK)
            def _():
                in_copy(k + 2, b).start()
        return carry

    lax.fori_loop(0, _NCHUNK // 2, superstep, 0)

    out_copy(_NCHUNK - 2, 0).wait()
    out_copy(_NCHUNK - 1, 1).wait()


def kernel(data, rois):
    rois_t = rois.T
    mesh = plsc.VectorSubcoreMesh(core_axis_name="c", subcore_axis_name="s")
    run = functools.partial(
        pl.kernel,
        mesh=mesh,
        out_type=jax.ShapeDtypeStruct((_H, _W, _N), jnp.float32),
        scratch_types=[
            pltpu.VMEM((_CW, _N), jnp.float32),
            pltpu.VMEM((_CW, _N), jnp.float32),
            pltpu.VMEM((_CW, _N), jnp.float32),
            pltpu.VMEM((_CW, _N), jnp.float32),
            pltpu.VMEM((4, _N), jnp.float32),
            pltpu.SemaphoreType.DMA,
            pltpu.SemaphoreType.DMA,
            pltpu.SemaphoreType.DMA,
            pltpu.SemaphoreType.DMA,
        ],
    )(_sc_body)
    return run(data, rois_t)


# speedup vs baseline: 1.4513x; 1.4513x over previous
import functools

import jax
import jax.numpy as jnp
from jax import lax
from jax.experimental import pallas as pl
from jax.experimental.pallas import tpu as pltpu
from jax.experimental.pallas import tpu_sc as plsc

_H, _W, _N = 512, 512, 100
_CW = 128
_CPP = _W // _CW
_PPW = _H // 32
_NCHUNK = _PPW * _CPP
_OFFS = (0, 16, 32, 48, 64, 80, 84)

_BIG = 1 << 20


def _sc_body(data_hbm, rois_hbm, out_hbm,
             inb0, inb1, outb0, outb1, roisv,
             isem0, isem1, osem0, osem1):
    nc = 2
    wid = lax.axis_index("s") * nc + lax.axis_index("c")
    p0 = wid * _PPW

    inb = (inb0, inb1)
    outb = (outb0, outb1)
    isem = (isem0, isem1)
    osem = (osem0, osem1)

    pltpu.sync_copy(rois_hbm, roisv)

    def chunk_plane(k):
        return p0 + k // _CPP

    def chunk_w0(k):
        return (k % _CPP) * _CW

    def in_copy(k, b):
        return pltpu.make_async_copy(
            data_hbm.at[chunk_plane(k), pl.ds(chunk_w0(k), _CW)],
            inb[b], isem[b])

    def out_copy(k, b):
        return pltpu.make_async_copy(
            outb[b], out_hbm.at[chunk_plane(k), pl.ds(chunk_w0(k), _CW)],
            osem[b])

    los = []
    rngs = []
    y1s = []
    y2s = []
    for g, off in enumerate(_OFFS):
        x1 = roisv[0, pl.ds(off, 16)]
        y1 = roisv[1, pl.ds(off, 16)]
        x2 = roisv[2, pl.ds(off, 16)]
        y2 = roisv[3, pl.ds(off, 16)]
        x1t = x1.astype(jnp.int32)
        lo = x1t + jnp.where(x1 > x1t.astype(jnp.float32), 1, 0)
        hi = x2.astype(jnp.int32)
        los.append(lo.astype(jnp.uint32))
        rngs.append(hi - lo)
        y1s.append(y1)
        y2s.append(y2)

    in_copy(0, 0).start()
    in_copy(1, 1).start()

    def superstep(s, carry):
        for b in range(2):
            k = s * 2 + b
            hf = chunk_plane(k).astype(jnp.float32)
            w0 = chunk_w0(k)

            lo_eff = []
            rng_eff = []
            for g in range(len(_OFFS)):
                act = ((hf >= y1s[g]) & (hf <= y2s[g])
                       & (rngs[g] >= 0))
                lo_eff.append(jnp.where(act, los[g], jnp.uint32(_BIG)))
                rng_eff.append(jnp.where(act, rngs[g].astype(jnp.uint32),
                                         jnp.uint32(0)))

            in_copy(k, b).wait()

            @pl.when(s >= 1)
            def _():
                out_copy(k - 2, b).wait()

            inref = inb[b]
            outref = outb[b]
            w0u = w0.astype(jnp.uint32)

            @plsc.parallel_loop(0, _CW, step=1, unroll=8)
            def _row(w):
                wg = w0u + w.astype(jnp.uint32)
                for g, off in enumerate(_OFFS):
                    d = inref[w, pl.ds(off, 16)]
                    m = (wg - lo_eff[g]) <= rng_eff[g]
                    outref[w, pl.ds(off, 16)] = jnp.where(m, d, 0.0)

            out_copy(k, b).start()

            @pl.when(k + 2 < _NCHUNK)
            def _():
                in_copy(k + 2, b).start()
        return carry

    lax.fori_loop(0, _NCHUNK // 2, superstep, 0)

    out_copy(_NCHUNK - 2, 0).wait()
    out_copy(_NCHUNK - 1, 1).wait()


def kernel(data, rois):
    rois_t = rois.T
    mesh = plsc.VectorSubcoreMesh(core_axis_name="c", subcore_axis_name="s")
    run = functools.partial(
        pl.kernel,
        mesh=mesh,
        out_type=jax.ShapeDtypeStruct((_H, _W, _N), jnp.float32),
        scratch_types=[
            pltpu.VMEM((_CW, _N), jnp.float32),
            pltpu.VMEM((_CW, _N), jnp.float32),
            pltpu.VMEM((_CW, _N), jnp.float32),
            pltpu.VMEM((_CW, _N), jnp.float32),
            pltpu.VMEM((4, _N), jnp.float32),
            pltpu.SemaphoreType.DMA,
            pltpu.SemaphoreType.DMA,
            pltpu.SemaphoreType.DMA,
            pltpu.SemaphoreType.DMA,
        ],
    )(_sc_body)
    return run(data, rois_t)
